# baseline (speedup 1.0000x reference)
import functools

import jax
import jax.numpy as jnp
from jax import lax
from jax.experimental import pallas as pl
from jax.experimental.pallas import tpu as pltpu
from jax.experimental.pallas import tpu_sc as plsc

_N = 20000
_K = 80
_W = 1333.0
_H = 800.0
_THRESH = 0.05
_NMS_T = 0.5
_TOPK = 100

_NFLAT = _N * _K
_NC = 2
_NS = 16
_NW = _NC * _NS
_SLICE = 51200
_BLK = 6400
_BCH = _BLK // 16
_GRP = _BCH // 16
_CAP = 2048
_C = _CAP * _NW
_ROWS = _C // 128


def _count_body(s_ref, o_ref):
    v = s_ref[...]
    mask = jnp.where(v > _THRESH, 1.0, 0.0).astype(jnp.float32)
    li = lax.broadcasted_iota(jnp.int32, (128, 8), 0)
    gi = lax.broadcasted_iota(jnp.int32, (128, 8), 1)
    bm = jnp.where(li // 16 == gi, 1.0, 0.0).astype(jnp.float32)
    o_ref[...] = jnp.dot(mask, bm, preferred_element_type=jnp.float32)


def _sc_compact(scores_flat, px1, py1, px2, py2, counts):
    mesh = plsc.VectorSubcoreMesh(core_axis_name="c", subcore_axis_name="s")

    @functools.partial(
        pl.kernel,
        mesh=mesh,
        out_type=[
            jax.ShapeDtypeStruct((_C,), jnp.float32),
            jax.ShapeDtypeStruct((_C,), jnp.int32),
            jax.ShapeDtypeStruct((_C,), jnp.float32),
            jax.ShapeDtypeStruct((_C,), jnp.float32),
            jax.ShapeDtypeStruct((_C,), jnp.float32),
            jax.ShapeDtypeStruct((_C,), jnp.float32),
        ],
        scratch_types=[
            pltpu.VMEM((_BLK,), jnp.float32),
            pltpu.VMEM((_BLK,), jnp.float32),
            pltpu.VMEM((_BLK,), jnp.float32),
            pltpu.VMEM((_BLK,), jnp.float32),
            pltpu.VMEM((_BLK,), jnp.float32),
            pltpu.VMEM((_BCH,), jnp.int32),
            pltpu.VMEM((_CAP + 16,), jnp.float32),
            pltpu.VMEM((_CAP + 16,), jnp.int32),
            pltpu.VMEM((_CAP + 16,), jnp.float32),
            pltpu.VMEM((_CAP + 16,), jnp.float32),
            pltpu.VMEM((_CAP + 16,), jnp.float32),
            pltpu.VMEM((_CAP + 16,), jnp.float32),
        ],
    )
    def sc_kernel(scores_hbm, x1_hbm, y1_hbm, x2_hbm, y2_hbm, cnt_hbm,
                  out_s, out_i, out_x1, out_y1, out_x2, out_y2,
                  bs, b1, b2, b3, b4, cb, cs, ci, c1, c2, c3, c4):
        wid = lax.axis_index("s") * _NC + lax.axis_index("c")
        base = wid * _SLICE
        nblk = jnp.where(wid < _NW - 1, jnp.int32(8), jnp.int32(2))

        neg16 = jnp.full((16,), -jnp.inf, jnp.float32)
        zero16f = jnp.zeros((16,), jnp.float32)
        zero16i = jnp.zeros((16,), jnp.int32)
        one = jnp.int32(1)
        zero = jnp.int32(0)

        def init_cap(j, carry):
            sl = pl.ds(j * 16, 16)
            cs[sl] = neg16
            ci[sl] = zero16i
            c1[sl] = zero16f
            c2[sl] = zero16f
            c3[sl] = zero16f
            c4[sl] = zero16f
            return carry

        lax.fori_loop(0, (_CAP + 16) // 16, init_cap, jnp.int32(0))

        def block(b, off):
            bbase = base + b * _BLK
            pltpu.sync_copy(scores_hbm.at[pl.ds(bbase, _BLK)], bs)
            pltpu.sync_copy(x1_hbm.at[pl.ds(bbase, _BLK)], b1)
            pltpu.sync_copy(y1_hbm.at[pl.ds(bbase, _BLK)], b2)
            pltpu.sync_copy(x2_hbm.at[pl.ds(bbase, _BLK)], b3)
            pltpu.sync_copy(y2_hbm.at[pl.ds(bbase, _BLK)], b4)
            cbase = wid * (_SLICE // 16) + b * _BCH
            pltpu.sync_copy(cnt_hbm.at[pl.ds(cbase, _BCH)], cb)

            def group(g, off):
                cl = cb[pl.ds(g * 16, 16)]
                for t in range(16):
                    j = g * 16 + t
                    cnt = cl[t]

                    @pl.when(cnt > 0)
                    def _appends(j=j, off=off):
                        sl = pl.ds(j * 16, 16)
                        v = bs[sl]
                        v1 = b1[sl]
                        v2 = b2[sl]
                        v3 = b3[sl]
                        v4 = b4[sl]
                        off0 = jnp.minimum(off, _CAP - 1)
                        acc = zero
                        for k in range(16):
                            dst = pl.ds(
                                jnp.minimum(off0 + acc, _CAP - 1), 16)
                            cs[dst] = zero16f + v[k]
                            ci[dst] = zero16i + (bbase + j * 16 + k)
                            c1[dst] = zero16f + v1[k]
                            c2[dst] = zero16f + v2[k]
                            c3[dst] = zero16f + v3[k]
                            c4[dst] = zero16f + v4[k]
                            acc = acc + jnp.where(v[k] > _THRESH, one, zero)

                    off = off + cnt
                return off

            return lax.fori_loop(0, _GRP, group, off)

        off_fin = lax.fori_loop(0, nblk, block, jnp.int32(0))
        cs[pl.ds(jnp.minimum(off_fin, _CAP - 1), 16)] = neg16

        obase = wid * _CAP
        osl = pl.ds(obase, _CAP)
        csl = pl.ds(0, _CAP)
        pltpu.sync_copy(cs.at[csl], out_s.at[osl])
        pltpu.sync_copy(ci.at[csl], out_i.at[osl])
        pltpu.sync_copy(c1.at[csl], out_x1.at[osl])
        pltpu.sync_copy(c2.at[csl], out_y1.at[osl])
        pltpu.sync_copy(c3.at[csl], out_x2.at[osl])
        pltpu.sync_copy(c4.at[csl], out_y2.at[osl])

    return sc_kernel(scores_flat, px1, py1, px2, py2, counts)


def _nms_body(sc_ref, idx_ref, x1_ref, y1_ref, x2_ref, y2_ref, out_ref):
    neg = jnp.float32(-jnp.inf)
    sc = sc_ref[...]
    idx = idx_ref[...]
    x1c = jnp.clip(x1_ref[...], 0.0, _W)
    y1c = jnp.clip(y1_ref[...], 0.0, _H)
    x2c = jnp.clip(x2_ref[...], 0.0, _W)
    y2c = jnp.clip(y2_ref[...], 0.0, _H)
    valid = sc > _THRESH
    clsf = (idx % _K).astype(jnp.float32)
    coordmax = jnp.maximum(jnp.maximum(x1c, y1c), jnp.maximum(x2c, y2c))
    mcoord = jnp.max(jnp.where(valid, coordmax, neg))
    offs = clsf * (mcoord + 1.0)
    x1 = x1c + offs
    y1 = y1c + offs
    x2 = x2c + offs
    y2 = y2c + offs
    areas = jnp.maximum(x2 - x1, 0.0) * jnp.maximum(y2 - y1, 0.0)

    rowi = lax.broadcasted_iota(jnp.int32, (_ROWS, 128), 0)
    coli = lax.broadcasted_iota(jnp.int32, (_ROWS, 128), 1)
    pos = rowi * 128 + coli
    base_s = jnp.where(valid, sc, neg)
    orow = lax.broadcasted_iota(jnp.int32, (8, 128), 0)
    ocol = lax.broadcasted_iota(jnp.int32, (8, 128), 1)
    zf = jnp.float32(0.0)

    def body(t, state):
        s, outbuf = state
        m = jnp.max(s)
        has = m > neg
        i = jnp.min(jnp.where(s == m, pos, jnp.int32(2**31 - 1)))
        sel = pos == i
        x1c_i = jnp.sum(jnp.where(sel, x1c, zf))
        y1c_i = jnp.sum(jnp.where(sel, y1c, zf))
        x2c_i = jnp.sum(jnp.where(sel, x2c, zf))
        y2c_i = jnp.sum(jnp.where(sel, y2c, zf))
        clsf_i = jnp.sum(jnp.where(sel, clsf, zf))
        offs_i = clsf_i * (mcoord + 1.0)
        x1_i = x1c_i + offs_i
        y1_i = y1c_i + offs_i
        x2_i = x2c_i + offs_i
        y2_i = y2c_i + offs_i
        area_i = (jnp.maximum(x2_i - x1_i, 0.0)
                  * jnp.maximum(y2_i - y1_i, 0.0))
        xx1 = jnp.maximum(x1_i, x1)
        yy1 = jnp.maximum(y1_i, y1)
        xx2 = jnp.minimum(x2_i, x2)
        yy2 = jnp.minimum(y2_i, y2)
        inter = jnp.maximum(xx2 - xx1, 0.0) * jnp.maximum(yy2 - yy1, 0.0)
        iou = inter / (area_i + areas - inter + 1e-9)
        kill = (iou > _NMS_T) | (pos == i)
        s = jnp.where(has & kill, neg, s)

        ox1 = jnp.where(has, x1c_i, zf)
        oy1 = jnp.where(has, y1c_i, zf)
        ox2 = jnp.where(has, x2c_i, zf)
        oy2 = jnp.where(has, y2c_i, zf)
        osc = jnp.where(has, m, zf)
        ocl = jnp.where(has, clsf_i, zf)
        newcol = jnp.where(orow == 0, ox1,
                 jnp.where(orow == 1, oy1,
                 jnp.where(orow == 2, ox2,
                 jnp.where(orow == 3, oy2,
                 jnp.where(orow == 4, osc, ocl)))))
        outbuf = jnp.where(ocol == t, newcol, outbuf)
        return s, outbuf

    out0 = jnp.zeros((8, 128), jnp.float32)
    _, outbuf = lax.fori_loop(0, _TOPK, body, (base_s, out0))
    out_ref[...] = outbuf


def kernel(boxes, scores):
    scores_flat = scores[:, :_K].reshape(-1)
    boxes_rows = boxes.reshape(_NFLAT, 4)
    px1 = boxes_rows[:, 0]
    py1 = boxes_rows[:, 1]
    px2 = boxes_rows[:, 2]
    py2 = boxes_rows[:, 3]

    counts_f = pl.pallas_call(
        _count_body,
        out_shape=jax.ShapeDtypeStruct((_NFLAT // 128, 8), jnp.float32),
    )(scores_flat.reshape(_NFLAT // 128, 128))
    counts = counts_f.astype(jnp.int32).reshape(-1)

    cand_s, cand_i, cx1, cy1, cx2, cy2 = _sc_compact(
        scores_flat, px1, py1, px2, py2, counts)

    packed = pl.pallas_call(
        _nms_body,
        out_shape=jax.ShapeDtypeStruct((8, 128), jnp.float32),
    )(cand_s.reshape(_ROWS, 128), cand_i.reshape(_ROWS, 128),
      cx1.reshape(_ROWS, 128), cy1.reshape(_ROWS, 128),
      cx2.reshape(_ROWS, 128), cy2.reshape(_ROWS, 128))

    out_boxes = packed[0:4, :_TOPK].T
    out_scores = packed[4, :_TOPK]
    out_classes = packed[5, :_TOPK].astype(jnp.int32)
    return out_boxes, out_scores, out_classes


# speedup vs baseline: 1.1620x; 1.1620x over previous
import functools

import jax
import jax.numpy as jnp
from jax import lax
from jax.experimental import pallas as pl
from jax.experimental.pallas import tpu as pltpu
from jax.experimental.pallas import tpu_sc as plsc

_N = 20000
_K = 80
_W = 1333.0
_H = 800.0
_THRESH = 0.05
_NMS_T = 0.5
_TOPK = 100

_NFLAT = _N * _K
_NC = 2
_NS = 16
_NW = _NC * _NS
_SLICE = _NFLAT // _NW
_BLK = 10000
_NBLK = _SLICE // _BLK
_BCH = _BLK // 16
_CAP = 2048
_PAD = 0
_C = _CAP * _NW
_ROWS = _C // 128


def _sc_compact(scores_flat, px1, py1, px2, py2):
    mesh = plsc.VectorSubcoreMesh(core_axis_name="c", subcore_axis_name="s")

    @functools.partial(
        pl.kernel,
        mesh=mesh,
        out_type=[
            jax.ShapeDtypeStruct((_C,), jnp.float32),
            jax.ShapeDtypeStruct((_C,), jnp.int32),
            jax.ShapeDtypeStruct((_C,), jnp.float32),
            jax.ShapeDtypeStruct((_C,), jnp.float32),
            jax.ShapeDtypeStruct((_C,), jnp.float32),
            jax.ShapeDtypeStruct((_C,), jnp.float32),
        ],
        scratch_types=[
            pltpu.VMEM((_BLK,), jnp.float32),
            pltpu.VMEM((_BLK,), jnp.float32),
            pltpu.VMEM((_BLK,), jnp.float32),
            pltpu.VMEM((_BLK,), jnp.float32),
            pltpu.VMEM((_BLK,), jnp.float32),
            pltpu.VMEM((_CAP + 16,), jnp.float32),
            pltpu.VMEM((_CAP + 16,), jnp.int32),
            pltpu.VMEM((_CAP + 16,), jnp.float32),
            pltpu.VMEM((_CAP + 16,), jnp.float32),
            pltpu.VMEM((_CAP + 16,), jnp.float32),
            pltpu.VMEM((_CAP + 16,), jnp.float32),
        ],
    )
    def sc_kernel(scores_hbm, x1_hbm, y1_hbm, x2_hbm, y2_hbm,
                  out_s, out_i, out_x1, out_y1, out_x2, out_y2,
                  bs, b1, b2, b3, b4, cs, ci, c1, c2, c3, c4):
        wid = lax.axis_index("s") * _NC + lax.axis_index("c")
        base = wid * _SLICE

        neg16 = jnp.full((16,), -jnp.inf, jnp.float32)
        zero16f = jnp.zeros((16,), jnp.float32)
        zero16i = jnp.zeros((16,), jnp.int32)

        def init_cap(j, carry):
            sl = pl.ds(j * 16, 16)
            cs[sl] = neg16
            ci[sl] = zero16i
            c1[sl] = zero16f
            c2[sl] = zero16f
            c3[sl] = zero16f
            c4[sl] = zero16f
            return carry

        lax.fori_loop(0, (_CAP + 16) // 16, init_cap, jnp.int32(0))

        def block(b, off):
            bbase = base + b * _BLK
            pltpu.sync_copy(scores_hbm.at[pl.ds(bbase, _BLK)], bs)
            pltpu.sync_copy(x1_hbm.at[pl.ds(bbase, _BLK)], b1)
            pltpu.sync_copy(y1_hbm.at[pl.ds(bbase, _BLK)], b2)
            pltpu.sync_copy(x2_hbm.at[pl.ds(bbase, _BLK)], b3)
            pltpu.sync_copy(y2_hbm.at[pl.ds(bbase, _BLK)], b4)

            one = jnp.int32(1)
            zero = jnp.int32(0)

            def chunk(j, off):
                sl = pl.ds(j * 16, 16)
                v = bs[sl]
                m = [v[k] > _THRESH for k in range(16)]
                mi = [jnp.where(mk, one, zero) for mk in m]
                pre = []
                acc = zero
                for k in range(16):
                    pre.append(acc)
                    acc = acc + mi[k]
                cnt = acc

                @pl.when(cnt > 0)
                def _appends():
                    v1 = b1[sl]
                    v2 = b2[sl]
                    v3 = b3[sl]
                    v4 = b4[sl]
                    off0 = jnp.minimum(off, _CAP - 1)
                    for k in range(16):
                        @pl.when(m[k])
                        def _append(k=k):
                            dst = pl.ds(
                                jnp.minimum(off0 + pre[k], _CAP - 1), 16)
                            cs[dst] = zero16f + v[k]
                            ci[dst] = zero16i + (bbase + j * 16 + k)
                            c1[dst] = zero16f + v1[k]
                            c2[dst] = zero16f + v2[k]
                            c3[dst] = zero16f + v3[k]
                            c4[dst] = zero16f + v4[k]

                return off + cnt

            return lax.fori_loop(0, _BCH, chunk, off)

        off_fin = lax.fori_loop(0, _NBLK, block, jnp.int32(0))
        cs[pl.ds(jnp.minimum(off_fin, _CAP - 1), 16)] = neg16

        obase = wid * _CAP
        osl = pl.ds(obase, _CAP)
        csl = pl.ds(0, _CAP)
        pltpu.sync_copy(cs.at[csl], out_s.at[osl])
        pltpu.sync_copy(ci.at[csl], out_i.at[osl])
        pltpu.sync_copy(c1.at[csl], out_x1.at[osl])
        pltpu.sync_copy(c2.at[csl], out_y1.at[osl])
        pltpu.sync_copy(c3.at[csl], out_x2.at[osl])
        pltpu.sync_copy(c4.at[csl], out_y2.at[osl])

    return sc_kernel(scores_flat, px1, py1, px2, py2)


def _nms_body(sc_ref, idx_ref, x1_ref, y1_ref, x2_ref, y2_ref, out_ref):
    neg = jnp.float32(-jnp.inf)
    sc = sc_ref[...]
    idx = idx_ref[...]
    x1c = jnp.clip(x1_ref[...], 0.0, _W)
    y1c = jnp.clip(y1_ref[...], 0.0, _H)
    x2c = jnp.clip(x2_ref[...], 0.0, _W)
    y2c = jnp.clip(y2_ref[...], 0.0, _H)
    valid = sc > _THRESH
    clsf = (idx % _K).astype(jnp.float32)
    coordmax = jnp.maximum(jnp.maximum(x1c, y1c), jnp.maximum(x2c, y2c))
    mcoord = jnp.max(jnp.where(valid, coordmax, neg))
    offs = clsf * (mcoord + 1.0)
    x1 = x1c + offs
    y1 = y1c + offs
    x2 = x2c + offs
    y2 = y2c + offs
    areas = jnp.maximum(x2 - x1, 0.0) * jnp.maximum(y2 - y1, 0.0)

    rowi = lax.broadcasted_iota(jnp.int32, (_ROWS, 128), 0)
    coli = lax.broadcasted_iota(jnp.int32, (_ROWS, 128), 1)
    pos = rowi * 128 + coli
    base_s = jnp.where(valid, sc, neg)
    orow = lax.broadcasted_iota(jnp.int32, (8, 128), 0)
    ocol = lax.broadcasted_iota(jnp.int32, (8, 128), 1)
    zf = jnp.float32(0.0)

    def body(t, state):
        s, outbuf = state
        m = jnp.max(s)
        has = m > neg
        i = jnp.min(jnp.where(s == m, pos, jnp.int32(2**31 - 1)))
        sel = pos == i
        x1c_i = jnp.sum(jnp.where(sel, x1c, zf))
        y1c_i = jnp.sum(jnp.where(sel, y1c, zf))
        x2c_i = jnp.sum(jnp.where(sel, x2c, zf))
        y2c_i = jnp.sum(jnp.where(sel, y2c, zf))
        clsf_i = jnp.sum(jnp.where(sel, clsf, zf))
        offs_i = clsf_i * (mcoord + 1.0)
        x1_i = x1c_i + offs_i
        y1_i = y1c_i + offs_i
        x2_i = x2c_i + offs_i
        y2_i = y2c_i + offs_i
        area_i = (jnp.maximum(x2_i - x1_i, 0.0)
                  * jnp.maximum(y2_i - y1_i, 0.0))
        xx1 = jnp.maximum(x1_i, x1)
        yy1 = jnp.maximum(y1_i, y1)
        xx2 = jnp.minimum(x2_i, x2)
        yy2 = jnp.minimum(y2_i, y2)
        inter = jnp.maximum(xx2 - xx1, 0.0) * jnp.maximum(yy2 - yy1, 0.0)
        iou = inter / (area_i + areas - inter + 1e-9)
        kill = (iou > _NMS_T) | (pos == i)
        s = jnp.where(has & kill, neg, s)

        ox1 = jnp.where(has, x1c_i, zf)
        oy1 = jnp.where(has, y1c_i, zf)
        ox2 = jnp.where(has, x2c_i, zf)
        oy2 = jnp.where(has, y2c_i, zf)
        osc = jnp.where(has, m, zf)
        ocl = jnp.where(has, clsf_i, zf)
        newcol = jnp.where(orow == 0, ox1,
                 jnp.where(orow == 1, oy1,
                 jnp.where(orow == 2, ox2,
                 jnp.where(orow == 3, oy2,
                 jnp.where(orow == 4, osc, ocl)))))
        outbuf = jnp.where(ocol == t, newcol, outbuf)
        return s, outbuf

    out0 = jnp.zeros((8, 128), jnp.float32)
    _, outbuf = lax.fori_loop(0, _TOPK, body, (base_s, out0))
    out_ref[...] = outbuf


def kernel(boxes, scores):
    scores_flat = scores[:, :_K].reshape(-1)
    boxes_rows = boxes.reshape(_NFLAT, 4)
    px1 = boxes_rows[:, 0]
    py1 = boxes_rows[:, 1]
    px2 = boxes_rows[:, 2]
    py2 = boxes_rows[:, 3]
    cand_s, cand_i, cx1, cy1, cx2, cy2 = _sc_compact(
        scores_flat, px1, py1, px2, py2)

    packed = pl.pallas_call(
        _nms_body,
        out_shape=jax.ShapeDtypeStruct((8, 128), jnp.float32),
    )(cand_s.reshape(_ROWS, 128), cand_i.reshape(_ROWS, 128),
      cx1.reshape(_ROWS, 128), cy1.reshape(_ROWS, 128),
      cx2.reshape(_ROWS, 128), cy2.reshape(_ROWS, 128))

    out_boxes = packed[0:4, :_TOPK].T
    out_scores = packed[4, :_TOPK]
    out_classes = packed[5, :_TOPK].astype(jnp.int32)
    return out_boxes, out_scores, out_classes
